# Initial kernel scaffold; baseline (speedup 1.0000x reference)
import jax
import jax.numpy as jnp
from jax.experimental import pallas as pl


def kernel(word_vectors, node_ids, positions, edge_index, edge_weight, graph_ids, pos, W_mlp, b_mlp, bn_gamma, bn_beta, att_w, att_b, pred_w, pred_b, pos_emb):
    raise NotImplementedError("write your pallas kernel here")


# speedup vs baseline: 2.6968x; 2.6968x over previous
import jax
import jax.numpy as jnp
from jax import lax
from jax.experimental import pallas as pl
from jax.experimental.pallas import tpu as pltpu
from jax.experimental.pallas import tpu_sc as plsc

_N_NODES = 10000
_N_EDGES = 320000
_D = 128
_N_GRAPHS = 64
_OUT_DIM = 16

_NC, _NS = 2, 16
_NW = _NC * _NS
_N_PAD = 10240
_ROWS_W = _N_PAD // _NW
_IDW = _ROWS_W // 64
_FEATS_W = _D // _NW
_FLAT_W = _FEATS_W * _N_PAD
_ECHUNK = 6400
_NCHUNK = _N_EDGES // _ECHUNK
_GROUPS = _ECHUNK // 16
_CB = 512
_NBLK = _N_PAD // _CB

_mesh = plsc.VectorSubcoreMesh(core_axis_name="c", subcore_axis_name="s",
                               num_cores=_NC, num_subcores=_NS)
_sc_params = pltpu.CompilerParams(needs_layout_passes=False)


def _wid():
    return lax.axis_index("s") * _NC + lax.axis_index("c")


def _sc_gather_body(wv_hbm, ids_hbm, pemb_hbm, posn_hbm, h_out, pe_out,
                    idx_v, rows_v, sem):
    w = _wid()
    base = pl.multiple_of(w * _ROWS_W, _ROWS_W)
    pltpu.sync_copy(ids_hbm.at[pl.ds(base, _ROWS_W)], idx_v)
    for j in range(_IDW):
        pltpu.async_copy(wv_hbm.at[idx_v.at[pl.ds(j * 64, 64)]],
                         rows_v.at[pl.ds(j * 64, 64)], sem).wait()
    pltpu.sync_copy(rows_v, h_out.at[pl.ds(base, _ROWS_W)])
    pltpu.sync_copy(posn_hbm.at[pl.ds(base, _ROWS_W)], idx_v)
    for j in range(_IDW):
        pltpu.async_copy(pemb_hbm.at[idx_v.at[pl.ds(j * 64, 64)]],
                         rows_v.at[pl.ds(j * 64, 64)], sem).wait()
    pltpu.sync_copy(rows_v, pe_out.at[pl.ds(base, _ROWS_W)])


def _sc_gather(word_vectors, ids2d, pos_emb, posn2d):
    f = pl.kernel(
        _sc_gather_body,
        out_type=[jax.ShapeDtypeStruct((_N_PAD, _D), jnp.float32),
                  jax.ShapeDtypeStruct((_N_PAD, _D), jnp.float32)],
        mesh=_mesh,
        scratch_types=[pltpu.VMEM((_ROWS_W,), jnp.int32),
                       pltpu.VMEM((_ROWS_W, _D), jnp.float32),
                       pltpu.SemaphoreType.DMA],
        compiler_params=_sc_params,
    )
    return f(word_vectors, ids2d, pos_emb, posn2d)


def _sc_edge_body(hin_hbm, pk_hbm, w_hbm, agg_out, hin_v, agg_v,
                  pk0, pk1, w0, w1, sem0, sem1):
    w = _wid()
    fbase = pl.multiple_of(w * _FLAT_W, _FLAT_W)
    pltpu.sync_copy(hin_hbm.at[pl.ds(fbase, _FLAT_W)], hin_v)

    zeros16 = jnp.zeros((16,), jnp.float32)

    def zbody(i, carry):
        agg_v[pl.ds(i * 16, 16)] = zeros16
        return carry

    lax.fori_loop(0, _FLAT_W // 16, zbody, 0, unroll=8)

    pks = [pk0, pk1]
    ws = [w0, w1]
    sems = [sem0, sem1]

    def start(ci, b):
        off = pl.multiple_of(ci * _ECHUNK, _ECHUNK)
        pltpu.async_copy(pk_hbm.at[pl.ds(off, _ECHUNK)], pks[b], sems[b])
        pltpu.async_copy(w_hbm.at[pl.ds(off, _ECHUNK)], ws[b], sems[b])

    def wait(ci, b):
        off = pl.multiple_of(ci * _ECHUNK, _ECHUNK)
        pltpu.make_async_copy(pk_hbm.at[pl.ds(off, _ECHUNK)], pks[b],
                              sems[b]).wait()
        pltpu.make_async_copy(w_hbm.at[pl.ds(off, _ECHUNK)], ws[b],
                              sems[b]).wait()

    def process(b):
        def group_body(g, gcarry):
            off = g * 16
            p = pks[b][pl.ds(off, 16)]
            ew = ws[b][pl.ds(off, 16)]
            col = jnp.bitwise_and(p, 16383)
            row = jnp.right_shift(p, 14)
            for f in range(_FEATS_W):
                vals = plsc.load_gather(hin_v, [col + f * _N_PAD])
                plsc.addupdate_scatter(agg_v, [row + f * _N_PAD], vals * ew)
            return gcarry

        lax.fori_loop(0, _GROUPS, group_body, 0, unroll=4)

    start(0, 0)

    def pair_body(cp, carry):
        for b in range(2):
            ci = cp * 2 + b
            wait(ci, b)

            @pl.when(ci + 1 < _NCHUNK)
            def _():
                start(ci + 1, 1 - b)

            process(b)
        return carry

    lax.fori_loop(0, _NCHUNK // 2, pair_body, 0)
    pltpu.sync_copy(agg_v, agg_out.at[pl.ds(fbase, _FLAT_W)])


def _sc_edge(hin_flat, packed, ew):
    f = pl.kernel(
        _sc_edge_body,
        out_type=jax.ShapeDtypeStruct((_D * _N_PAD,), jnp.float32),
        mesh=_mesh,
        scratch_types=[pltpu.VMEM((_FLAT_W,), jnp.float32),
                       pltpu.VMEM((_FLAT_W,), jnp.float32),
                       pltpu.VMEM((_ECHUNK,), jnp.int32),
                       pltpu.VMEM((_ECHUNK,), jnp.int32),
                       pltpu.VMEM((_ECHUNK,), jnp.float32),
                       pltpu.VMEM((_ECHUNK,), jnp.float32),
                       pltpu.SemaphoreType.DMA,
                       pltpu.SemaphoreType.DMA],
        compiler_params=_sc_params,
    )
    return f(hin_flat, packed, ew)


def _tc0_body(pos_ref, h_ref, pe_ref, h0t_ref, hin0t_ref, pet_ref):
    ht = h_ref[...].T
    pet = pe_ref[...].T
    pet_ref[...] = pet
    h0t_ref[...] = ht + pet
    hin0t_ref[...] = ht + pos_ref[0] * pet


def _tc0(pos, h, pe):
    return pl.pallas_call(
        _tc0_body,
        grid=(_NBLK,),
        in_specs=[pl.BlockSpec(memory_space=pltpu.SMEM),
                  pl.BlockSpec((_CB, _D), lambda i: (i, 0)),
                  pl.BlockSpec((_CB, _D), lambda i: (i, 0))],
        out_specs=[pl.BlockSpec((_D, _CB), lambda i: (0, i))] * 3,
        out_shape=[jax.ShapeDtypeStruct((_D, _N_PAD), jnp.float32)] * 3,
    )(pos, h, pe)


def _mlp(agg, w1t_ref, b1_ref, w2t_ref, b2_ref, scale_ref, beta_ref):
    x = jnp.dot(w1t_ref[...], agg, preferred_element_type=jnp.float32)
    x = jnp.maximum(x + b1_ref[...], 0.0)
    x = jnp.dot(w2t_ref[...], x, preferred_element_type=jnp.float32)
    x = (x + b2_ref[...]) * scale_ref[...] + beta_ref[...]
    return jnp.where(x >= 0.0, x, 0.01 * x)


def _tc1_body(pos_ref, agg_ref, pe_ref, w1t_ref, b1_ref, w2t_ref, b2_ref,
              scale_ref, beta_ref, h1t_ref, hin1t_ref):
    h = _mlp(agg_ref[...], w1t_ref, b1_ref, w2t_ref, b2_ref, scale_ref,
             beta_ref)
    h1t_ref[...] = h
    hin1t_ref[...] = h + pos_ref[0] * pe_ref[...]


def _tc1(pos1, aggt, pet, w1t, b1, w2t, b2, scale, beta):
    wspec = pl.BlockSpec((_D, _D), lambda i: (0, 0))
    bspec = pl.BlockSpec((_D, 1), lambda i: (0, 0))
    return pl.pallas_call(
        _tc1_body,
        grid=(_NBLK,),
        in_specs=[pl.BlockSpec(memory_space=pltpu.SMEM),
                  pl.BlockSpec((_D, _CB), lambda i: (0, i)),
                  pl.BlockSpec((_D, _CB), lambda i: (0, i)),
                  wspec, bspec, wspec, bspec, bspec, bspec],
        out_specs=[pl.BlockSpec((_D, _CB), lambda i: (0, i))] * 2,
        out_shape=[jax.ShapeDtypeStruct((_D, _N_PAD), jnp.float32)] * 2,
    )(pos1, aggt, pet, w1t, b1, w2t, b2, scale, beta)


def _tc2_body(attgp_ref, attb_ref, agg_ref, h0_ref, h1_ref, gid_ref,
              w1t_ref, b1_ref, w2t_ref, b2_ref, scale_ref, beta_ref,
              attw_ref, p0, p1, p2, r0, r1, r2):
    i = pl.program_id(0)
    pouts = [p0, p1, p2]
    routs = [r0, r1, r2]

    @pl.when(i == 0)
    def _():
        for l in range(3):
            pouts[l][...] = jnp.zeros((_D, _N_GRAPHS), jnp.float32)
            routs[l][...] = jnp.zeros((1, _N_GRAPHS), jnp.float32)

    h2 = _mlp(agg_ref[...], w1t_ref, b1_ref, w2t_ref, b2_ref, scale_ref,
              beta_ref)
    hs = [h0_ref[...], h1_ref[...], h2]
    gid = gid_ref[...]
    giota = lax.broadcasted_iota(jnp.int32, (_CB, _N_GRAPHS), 1)
    niota = lax.broadcasted_iota(jnp.int32, (_CB, 1), 0) + i * _CB
    onehot = jnp.where((gid == giota) & (niota < _N_NODES), 1.0, 0.0)
    e = jnp.ones((1, _CB), jnp.float32)
    for l in range(3):
        hl = hs[l]
        logits = (jnp.sum(hl * attw_ref[:, l:l + 1], axis=0, keepdims=True)
                  + e * attgp_ref[l] + attb_ref[l])
        lk = jnp.where(logits >= 0.0, logits, 0.01 * logits)
        e = jnp.exp(lk * (-1.0 / 20.0))
        pouts[l][...] += jnp.dot(hl * e, onehot,
                                 preferred_element_type=jnp.float32)
        routs[l][...] += jnp.dot(e, onehot,
                                 preferred_element_type=jnp.float32)


def _tc2(attgp, attb, aggt, h0t, h1t, gid2d, w1t, b1, w2t, b2, scale, beta,
         attw):
    wspec = pl.BlockSpec((_D, _D), lambda i: (0, 0))
    bspec = pl.BlockSpec((_D, 1), lambda i: (0, 0))
    cspec = pl.BlockSpec((_D, _CB), lambda i: (0, i))
    return pl.pallas_call(
        _tc2_body,
        grid=(_NBLK,),
        in_specs=[pl.BlockSpec(memory_space=pltpu.SMEM),
                  pl.BlockSpec(memory_space=pltpu.SMEM),
                  cspec, cspec, cspec,
                  pl.BlockSpec((_CB, 1), lambda i: (i, 0)),
                  wspec, bspec, wspec, bspec, bspec, bspec,
                  pl.BlockSpec((_D, 3), lambda i: (0, 0))],
        out_specs=[pl.BlockSpec((_D, _N_GRAPHS), lambda i: (0, 0))] * 3
        + [pl.BlockSpec((1, _N_GRAPHS), lambda i: (0, 0))] * 3,
        out_shape=[jax.ShapeDtypeStruct((_D, _N_GRAPHS), jnp.float32)] * 3
        + [jax.ShapeDtypeStruct((1, _N_GRAPHS), jnp.float32)] * 3,
    )(attgp, attb, aggt, h0t, h1t, gid2d, w1t, b1, w2t, b2, scale, beta, attw)


def _tcf_body(p0, p1, p2, r0, r1, r2, predwt_ref, predb_ref, score_ref):
    acc = jnp.broadcast_to(predb_ref[...], (_OUT_DIM, _N_GRAPHS))
    ps = [p0, p1, p2]
    rs = [r0, r1, r2]
    for l in range(3):
        pooled = ps[l][...] / rs[l][...]
        acc = acc + jnp.dot(predwt_ref[l], pooled,
                            preferred_element_type=jnp.float32)
    score_ref[...] = acc.T


def _tcf(p0, p1, p2, r0, r1, r2, predwt, predb_sum):
    return pl.pallas_call(
        _tcf_body,
        out_shape=jax.ShapeDtypeStruct((_N_GRAPHS, _OUT_DIM), jnp.float32),
    )(p0, p1, p2, r0, r1, r2, predwt, predb_sum)


def kernel(word_vectors, node_ids, positions, edge_index, edge_weight,
           graph_ids, pos, W_mlp, b_mlp, bn_gamma, bn_beta, att_w, att_b,
           pred_w, pred_b, pos_emb):
    f32 = jnp.float32
    npad = _N_PAD - _N_NODES
    ids1d = jnp.concatenate(
        [node_ids.astype(jnp.int32), jnp.zeros((npad,), jnp.int32)])
    posn1d = jnp.concatenate(
        [positions.astype(jnp.int32), jnp.zeros((npad,), jnp.int32)])
    packed = jnp.bitwise_or(
        jnp.left_shift(edge_index[0].astype(jnp.int32), 14),
        edge_index[1].astype(jnp.int32))
    gid2d = jnp.concatenate(
        [graph_ids.astype(jnp.int32),
         jnp.full((npad,), _N_GRAPHS - 1, jnp.int32)]
    ).reshape(_N_PAD, 1)

    inv = 1.0 / jnp.sqrt(jnp.asarray(1.0 + 1e-5, f32))
    w1t = [W_mlp[l, 0].T for l in range(2)]
    w2t = [W_mlp[l, 1].T for l in range(2)]
    b1 = [b_mlp[l, 0][:, None] for l in range(2)]
    b2 = [b_mlp[l, 1][:, None] for l in range(2)]
    scale = [(bn_gamma[l] * inv)[:, None] for l in range(2)]
    beta = [bn_beta[l][:, None] for l in range(2)]
    attw = att_w[:, :_D, 0].T
    attgp = att_w[:, _D, 0]
    attb = att_b[:, 0]
    predwt = jnp.transpose(pred_w, (0, 2, 1))
    predb_sum = jnp.sum(pred_b, axis=0)[:, None]

    h, pe = _sc_gather(word_vectors, ids1d, pos_emb, posn1d)
    h0t, hin0t, pet = _tc0(pos, h, pe)
    agg0t = _sc_edge(hin0t.reshape(-1), packed, edge_weight).reshape(_D, _N_PAD)
    h1t, hin1t = _tc1(pos[1:2], agg0t, pet, w1t[0], b1[0], w2t[0], b2[0],
                      scale[0], beta[0])
    agg1t = _sc_edge(hin1t.reshape(-1), packed, edge_weight).reshape(_D, _N_PAD)
    p0, p1, p2, r0, r1, r2 = _tc2(attgp, attb, agg1t, h0t, h1t, gid2d,
                                  w1t[1], b1[1], w2t[1], b2[1], scale[1],
                                  beta[1], attw)
    return _tcf(p0, p1, p2, r0, r1, r2, predwt, predb_sum)


# speedup vs baseline: 6.8714x; 2.5480x over previous
import jax
import jax.numpy as jnp
from jax import lax
from jax.experimental import pallas as pl
from jax.experimental.pallas import tpu as pltpu
from jax.experimental.pallas import tpu_sc as plsc

_N_NODES = 10000
_N_EDGES = 320000
_D = 128
_N_GRAPHS = 64
_OUT_DIM = 16

_NC, _NS = 2, 16
_NW = _NC * _NS
_N_PAD = 10240
_ROWS_W = _N_PAD // _NW
_IDW = _ROWS_W // 64
_FEATS_W = _D // _NW
_FLAT_W = _FEATS_W * _N_PAD
_ECHUNK = 6400
_NCHUNK = _N_EDGES // _ECHUNK
_GROUPS = _ECHUNK // 16
_CB = 512
_NBLK = _N_PAD // _CB

_mesh = plsc.VectorSubcoreMesh(core_axis_name="c", subcore_axis_name="s",
                               num_cores=_NC, num_subcores=_NS)
_sc_params = pltpu.CompilerParams(needs_layout_passes=False,
                                  disable_bounds_checks=True)


def _wid():
    return lax.axis_index("s") * _NC + lax.axis_index("c")


def _sc_gather_body(wv_hbm, ids_hbm, pemb_hbm, posn_hbm, h_out, pe_out,
                    idx_v, rows_v, sem):
    w = _wid()
    base = pl.multiple_of(w * _ROWS_W, _ROWS_W)
    pltpu.sync_copy(ids_hbm.at[pl.ds(base, _ROWS_W)], idx_v)
    for j in range(_IDW):
        pltpu.async_copy(wv_hbm.at[idx_v.at[pl.ds(j * 64, 64)]],
                         rows_v.at[pl.ds(j * 64, 64)], sem).wait()
    pltpu.sync_copy(rows_v, h_out.at[pl.ds(base, _ROWS_W)])
    pltpu.sync_copy(posn_hbm.at[pl.ds(base, _ROWS_W)], idx_v)
    for j in range(_IDW):
        pltpu.async_copy(pemb_hbm.at[idx_v.at[pl.ds(j * 64, 64)]],
                         rows_v.at[pl.ds(j * 64, 64)], sem).wait()
    pltpu.sync_copy(rows_v, pe_out.at[pl.ds(base, _ROWS_W)])


def _sc_gather(word_vectors, ids2d, pos_emb, posn2d):
    f = pl.kernel(
        _sc_gather_body,
        out_type=[jax.ShapeDtypeStruct((_N_PAD, _D), jnp.float32),
                  jax.ShapeDtypeStruct((_N_PAD, _D), jnp.float32)],
        mesh=_mesh,
        scratch_types=[pltpu.VMEM((_ROWS_W,), jnp.int32),
                       pltpu.VMEM((_ROWS_W, _D), jnp.float32),
                       pltpu.SemaphoreType.DMA],
        compiler_params=_sc_params,
    )
    return f(word_vectors, ids2d, pos_emb, posn2d)


def _sc_edge_body(hin_hbm, pk_hbm, w_hbm, agg_out, hin_v, agg_v,
                  pk0, pk1, w0, w1, sem0, sem1):
    w = _wid()
    fbase = pl.multiple_of(w * _FLAT_W, _FLAT_W)
    pltpu.sync_copy(hin_hbm.at[pl.ds(fbase, _FLAT_W)], hin_v)

    zeros16 = jnp.zeros((16,), jnp.float32)

    @plsc.parallel_loop(0, _FLAT_W, 16, unroll=8)
    def _(i):
        agg_v[pl.ds(i, 16)] = zeros16

    pks = [pk0, pk1]
    ws = [w0, w1]
    sems = [sem0, sem1]

    def start(ci, b):
        off = pl.multiple_of(ci * _ECHUNK, _ECHUNK)
        pltpu.async_copy(pk_hbm.at[pl.ds(off, _ECHUNK)], pks[b], sems[b])
        pltpu.async_copy(w_hbm.at[pl.ds(off, _ECHUNK)], ws[b], sems[b])

    def wait(ci, b):
        off = pl.multiple_of(ci * _ECHUNK, _ECHUNK)
        pltpu.make_async_copy(pk_hbm.at[pl.ds(off, _ECHUNK)], pks[b],
                              sems[b]).wait()
        pltpu.make_async_copy(w_hbm.at[pl.ds(off, _ECHUNK)], ws[b],
                              sems[b]).wait()

    def process(b):
        @plsc.parallel_loop(0, _ECHUNK, 16, unroll=8)
        def _(off):
            p = pks[b][pl.ds(off, 16)]
            ew = ws[b][pl.ds(off, 16)]
            col = jnp.bitwise_and(p, 16383)
            row = jnp.right_shift(p, 14)
            for f in range(_FEATS_W):
                vals = plsc.load_gather(hin_v, [col + f * _N_PAD])
                plsc.addupdate_scatter(agg_v, [row + f * _N_PAD], vals * ew)

    start(0, 0)

    def pair_body(cp, carry):
        for b in range(2):
            ci = cp * 2 + b
            wait(ci, b)

            @pl.when(ci + 1 < _NCHUNK)
            def _():
                start(ci + 1, 1 - b)

            process(b)
        return carry

    lax.fori_loop(0, _NCHUNK // 2, pair_body, 0)
    pltpu.sync_copy(agg_v, agg_out.at[pl.ds(fbase, _FLAT_W)])


def _sc_edge(hin_flat, packed, ew):
    f = pl.kernel(
        _sc_edge_body,
        out_type=jax.ShapeDtypeStruct((_D * _N_PAD,), jnp.float32),
        mesh=_mesh,
        scratch_types=[pltpu.VMEM((_FLAT_W,), jnp.float32),
                       pltpu.VMEM((_FLAT_W,), jnp.float32),
                       pltpu.VMEM((_ECHUNK,), jnp.int32),
                       pltpu.VMEM((_ECHUNK,), jnp.int32),
                       pltpu.VMEM((_ECHUNK,), jnp.float32),
                       pltpu.VMEM((_ECHUNK,), jnp.float32),
                       pltpu.SemaphoreType.DMA,
                       pltpu.SemaphoreType.DMA],
        compiler_params=_sc_params,
    )
    return f(hin_flat, packed, ew)


def _tc0_body(pos_ref, h_ref, pe_ref, h0t_ref, hin0t_ref, pet_ref):
    ht = h_ref[...].T
    pet = pe_ref[...].T
    pet_ref[...] = pet
    h0t_ref[...] = ht + pet
    hin0t_ref[...] = ht + pos_ref[0] * pet


def _tc0(pos, h, pe):
    return pl.pallas_call(
        _tc0_body,
        grid=(_NBLK,),
        in_specs=[pl.BlockSpec(memory_space=pltpu.SMEM),
                  pl.BlockSpec((_CB, _D), lambda i: (i, 0)),
                  pl.BlockSpec((_CB, _D), lambda i: (i, 0))],
        out_specs=[pl.BlockSpec((_D, _CB), lambda i: (0, i))] * 3,
        out_shape=[jax.ShapeDtypeStruct((_D, _N_PAD), jnp.float32)] * 3,
    )(pos, h, pe)


def _mlp(agg, w1t_ref, b1_ref, w2t_ref, b2_ref, scale_ref, beta_ref):
    x = jnp.dot(w1t_ref[...], agg, preferred_element_type=jnp.float32)
    x = jnp.maximum(x + b1_ref[...], 0.0)
    x = jnp.dot(w2t_ref[...], x, preferred_element_type=jnp.float32)
    x = (x + b2_ref[...]) * scale_ref[...] + beta_ref[...]
    return jnp.where(x >= 0.0, x, 0.01 * x)


def _tc1_body(pos_ref, agg_ref, pe_ref, w1t_ref, b1_ref, w2t_ref, b2_ref,
              scale_ref, beta_ref, h1t_ref, hin1t_ref):
    h = _mlp(agg_ref[...], w1t_ref, b1_ref, w2t_ref, b2_ref, scale_ref,
             beta_ref)
    h1t_ref[...] = h
    hin1t_ref[...] = h + pos_ref[0] * pe_ref[...]


def _tc1(pos1, aggt, pet, w1t, b1, w2t, b2, scale, beta):
    wspec = pl.BlockSpec((_D, _D), lambda i: (0, 0))
    bspec = pl.BlockSpec((_D, 1), lambda i: (0, 0))
    return pl.pallas_call(
        _tc1_body,
        grid=(_NBLK,),
        in_specs=[pl.BlockSpec(memory_space=pltpu.SMEM),
                  pl.BlockSpec((_D, _CB), lambda i: (0, i)),
                  pl.BlockSpec((_D, _CB), lambda i: (0, i)),
                  wspec, bspec, wspec, bspec, bspec, bspec],
        out_specs=[pl.BlockSpec((_D, _CB), lambda i: (0, i))] * 2,
        out_shape=[jax.ShapeDtypeStruct((_D, _N_PAD), jnp.float32)] * 2,
    )(pos1, aggt, pet, w1t, b1, w2t, b2, scale, beta)


def _tc2_body(attgp_ref, attb_ref, agg_ref, h0_ref, h1_ref, gid_ref,
              w1t_ref, b1_ref, w2t_ref, b2_ref, scale_ref, beta_ref,
              attw_ref, p0, p1, p2, r0, r1, r2):
    i = pl.program_id(0)
    pouts = [p0, p1, p2]
    routs = [r0, r1, r2]

    @pl.when(i == 0)
    def _():
        for l in range(3):
            pouts[l][...] = jnp.zeros((_D, _N_GRAPHS), jnp.float32)
            routs[l][...] = jnp.zeros((1, _N_GRAPHS), jnp.float32)

    h2 = _mlp(agg_ref[...], w1t_ref, b1_ref, w2t_ref, b2_ref, scale_ref,
              beta_ref)
    hs = [h0_ref[...], h1_ref[...], h2]
    gid = gid_ref[...]
    giota = lax.broadcasted_iota(jnp.int32, (_CB, _N_GRAPHS), 1)
    niota = lax.broadcasted_iota(jnp.int32, (_CB, 1), 0) + i * _CB
    onehot = jnp.where((gid == giota) & (niota < _N_NODES), 1.0, 0.0)
    e = jnp.ones((1, _CB), jnp.float32)
    for l in range(3):
        hl = hs[l]
        logits = (jnp.sum(hl * attw_ref[:, l:l + 1], axis=0, keepdims=True)
                  + e * attgp_ref[l] + attb_ref[l])
        lk = jnp.where(logits >= 0.0, logits, 0.01 * logits)
        e = jnp.exp(lk * (-1.0 / 20.0))
        pouts[l][...] += jnp.dot(hl * e, onehot,
                                 preferred_element_type=jnp.float32)
        routs[l][...] += jnp.dot(e, onehot,
                                 preferred_element_type=jnp.float32)


def _tc2(attgp, attb, aggt, h0t, h1t, gid2d, w1t, b1, w2t, b2, scale, beta,
         attw):
    wspec = pl.BlockSpec((_D, _D), lambda i: (0, 0))
    bspec = pl.BlockSpec((_D, 1), lambda i: (0, 0))
    cspec = pl.BlockSpec((_D, _CB), lambda i: (0, i))
    return pl.pallas_call(
        _tc2_body,
        grid=(_NBLK,),
        in_specs=[pl.BlockSpec(memory_space=pltpu.SMEM),
                  pl.BlockSpec(memory_space=pltpu.SMEM),
                  cspec, cspec, cspec,
                  pl.BlockSpec((_CB, 1), lambda i: (i, 0)),
                  wspec, bspec, wspec, bspec, bspec, bspec,
                  pl.BlockSpec((_D, 3), lambda i: (0, 0))],
        out_specs=[pl.BlockSpec((_D, _N_GRAPHS), lambda i: (0, 0))] * 3
        + [pl.BlockSpec((1, _N_GRAPHS), lambda i: (0, 0))] * 3,
        out_shape=[jax.ShapeDtypeStruct((_D, _N_GRAPHS), jnp.float32)] * 3
        + [jax.ShapeDtypeStruct((1, _N_GRAPHS), jnp.float32)] * 3,
    )(attgp, attb, aggt, h0t, h1t, gid2d, w1t, b1, w2t, b2, scale, beta, attw)


def _tcf_body(p0, p1, p2, r0, r1, r2, predwt_ref, predb_ref, score_ref):
    acc = jnp.broadcast_to(predb_ref[...], (_OUT_DIM, _N_GRAPHS))
    ps = [p0, p1, p2]
    rs = [r0, r1, r2]
    for l in range(3):
        pooled = ps[l][...] / rs[l][...]
        acc = acc + jnp.dot(predwt_ref[l], pooled,
                            preferred_element_type=jnp.float32)
    score_ref[...] = acc.T


def _tcf(p0, p1, p2, r0, r1, r2, predwt, predb_sum):
    return pl.pallas_call(
        _tcf_body,
        out_shape=jax.ShapeDtypeStruct((_N_GRAPHS, _OUT_DIM), jnp.float32),
    )(p0, p1, p2, r0, r1, r2, predwt, predb_sum)


def kernel(word_vectors, node_ids, positions, edge_index, edge_weight,
           graph_ids, pos, W_mlp, b_mlp, bn_gamma, bn_beta, att_w, att_b,
           pred_w, pred_b, pos_emb):
    f32 = jnp.float32
    npad = _N_PAD - _N_NODES
    ids1d = jnp.concatenate(
        [node_ids.astype(jnp.int32), jnp.zeros((npad,), jnp.int32)])
    posn1d = jnp.concatenate(
        [positions.astype(jnp.int32), jnp.zeros((npad,), jnp.int32)])
    packed = jnp.bitwise_or(
        jnp.left_shift(edge_index[0].astype(jnp.int32), 14),
        edge_index[1].astype(jnp.int32))
    gid2d = jnp.concatenate(
        [graph_ids.astype(jnp.int32),
         jnp.full((npad,), _N_GRAPHS - 1, jnp.int32)]
    ).reshape(_N_PAD, 1)

    inv = 1.0 / jnp.sqrt(jnp.asarray(1.0 + 1e-5, f32))
    w1t = [W_mlp[l, 0].T for l in range(2)]
    w2t = [W_mlp[l, 1].T for l in range(2)]
    b1 = [b_mlp[l, 0][:, None] for l in range(2)]
    b2 = [b_mlp[l, 1][:, None] for l in range(2)]
    scale = [(bn_gamma[l] * inv)[:, None] for l in range(2)]
    beta = [bn_beta[l][:, None] for l in range(2)]
    attw = att_w[:, :_D, 0].T
    attgp = att_w[:, _D, 0]
    attb = att_b[:, 0]
    predwt = jnp.transpose(pred_w, (0, 2, 1))
    predb_sum = jnp.sum(pred_b, axis=0)[:, None]

    h, pe = _sc_gather(word_vectors, ids1d, pos_emb, posn1d)
    h0t, hin0t, pet = _tc0(pos, h, pe)
    agg0t = _sc_edge(hin0t.reshape(-1), packed, edge_weight).reshape(_D, _N_PAD)
    h1t, hin1t = _tc1(pos[1:2], agg0t, pet, w1t[0], b1[0], w2t[0], b2[0],
                      scale[0], beta[0])
    agg1t = _sc_edge(hin1t.reshape(-1), packed, edge_weight).reshape(_D, _N_PAD)
    p0, p1, p2, r0, r1, r2 = _tc2(attgp, attb, agg1t, h0t, h1t, gid2d,
                                  w1t[1], b1[1], w2t[1], b2[1], scale[1],
                                  beta[1], attw)
    return _tcf(p0, p1, p2, r0, r1, r2, predwt, predb_sum)


# speedup vs baseline: 7.8922x; 1.1486x over previous
import jax
import jax.numpy as jnp
from jax import lax
from jax.experimental import pallas as pl
from jax.experimental.pallas import tpu as pltpu
from jax.experimental.pallas import tpu_sc as plsc

_N_NODES = 10000
_N_EDGES = 320000
_D = 128
_N_GRAPHS = 64
_OUT_DIM = 16

_NC, _NS = 2, 16
_NW = _NC * _NS
_N_PAD = 10240
_ROWS_W = _N_PAD // _NW
_IDW = _ROWS_W // 64
_FEATS_W = _D // _NW
_FLAT_W = _FEATS_W * _N_PAD
_ECHUNK = 6400
_NCHUNK = _N_EDGES // _ECHUNK
_GROUPS = _ECHUNK // 16
_CB = 512
_NBLK = _N_PAD // _CB

_mesh = plsc.VectorSubcoreMesh(core_axis_name="c", subcore_axis_name="s",
                               num_cores=_NC, num_subcores=_NS)
_sc_params = pltpu.CompilerParams(needs_layout_passes=False,
                                  disable_bounds_checks=True)


def _wid():
    return lax.axis_index("s") * _NC + lax.axis_index("c")


def _sc_gather_body(wv_hbm, ids_hbm, pemb_hbm, posn_hbm, h_out, pe_out,
                    idx_h, idx_p, rows_h, rows_p, sem0, sem1):
    w = _wid()
    base = pl.multiple_of(w * _ROWS_W, _ROWS_W)
    pltpu.sync_copy(ids_hbm.at[pl.ds(base, _ROWS_W)], idx_h)
    pltpu.sync_copy(posn_hbm.at[pl.ds(base, _ROWS_W)], idx_p)
    for j in range(_IDW):
        pltpu.async_copy(wv_hbm.at[idx_h.at[pl.ds(j * 64, 64)]],
                         rows_h.at[pl.ds(j * 64, 64)], sem0)
    for j in range(_IDW):
        pltpu.async_copy(pemb_hbm.at[idx_p.at[pl.ds(j * 64, 64)]],
                         rows_p.at[pl.ds(j * 64, 64)], sem1)
    for j in range(_IDW):
        pltpu.make_async_copy(wv_hbm.at[idx_h.at[pl.ds(j * 64, 64)]],
                              rows_h.at[pl.ds(j * 64, 64)], sem0).wait()
    pltpu.sync_copy(rows_h, h_out.at[pl.ds(base, _ROWS_W)])
    for j in range(_IDW):
        pltpu.make_async_copy(pemb_hbm.at[idx_p.at[pl.ds(j * 64, 64)]],
                              rows_p.at[pl.ds(j * 64, 64)], sem1).wait()
    pltpu.sync_copy(rows_p, pe_out.at[pl.ds(base, _ROWS_W)])


def _sc_gather(word_vectors, ids2d, pos_emb, posn2d):
    f = pl.kernel(
        _sc_gather_body,
        out_type=[jax.ShapeDtypeStruct((_N_PAD, _D), jnp.float32),
                  jax.ShapeDtypeStruct((_N_PAD, _D), jnp.float32)],
        mesh=_mesh,
        scratch_types=[pltpu.VMEM((_ROWS_W,), jnp.int32),
                       pltpu.VMEM((_ROWS_W,), jnp.int32),
                       pltpu.VMEM((_ROWS_W, _D), jnp.float32),
                       pltpu.VMEM((_ROWS_W, _D), jnp.float32),
                       pltpu.SemaphoreType.DMA,
                       pltpu.SemaphoreType.DMA],
        compiler_params=_sc_params,
    )
    return f(word_vectors, ids2d, pos_emb, posn2d)


def _sc_edge_body(hin_hbm, pk_hbm, w_hbm, agg_out, h0, h1,
                  a0, a1, a2, a3, pk0, pk1, w0, w1, sem0, sem1):
    w = _wid()
    hins = [h0, h1]
    aggs = [a0, a1, a2, a3]
    for k in range(2):
        off = pl.multiple_of((w * 2 + k) * _N_PAD, _N_PAD)
        pltpu.sync_copy(hin_hbm.at[pl.ds(off, _N_PAD)], hins[k])

    zeros16 = jnp.zeros((16,), jnp.float32)

    @plsc.parallel_loop(0, _N_PAD, 16, unroll=8)
    def _(i):
        for f in range(_FEATS_W):
            aggs[f][pl.ds(i, 16)] = zeros16

    pks = [pk0, pk1]
    ws = [w0, w1]
    sems = [sem0, sem1]

    def start(ci, b):
        off = pl.multiple_of(ci * _ECHUNK, _ECHUNK)
        pltpu.async_copy(pk_hbm.at[pl.ds(off, _ECHUNK)], pks[b], sems[b])
        pltpu.async_copy(w_hbm.at[pl.ds(off, _ECHUNK)], ws[b], sems[b])

    def wait(ci, b):
        off = pl.multiple_of(ci * _ECHUNK, _ECHUNK)
        pltpu.make_async_copy(pk_hbm.at[pl.ds(off, _ECHUNK)], pks[b],
                              sems[b]).wait()
        pltpu.make_async_copy(w_hbm.at[pl.ds(off, _ECHUNK)], ws[b],
                              sems[b]).wait()

    def process(b):
        @plsc.parallel_loop(0, _ECHUNK, 16, unroll=8)
        def _(off):
            p = pks[b][pl.ds(off, 16)]
            ew = ws[b][pl.ds(off, 16)]
            col = jnp.bitwise_and(p, 16383)
            row = jnp.right_shift(p, 14)
            for k in range(2):
                g = plsc.load_gather(hins[k], [col])
                gb = plsc.bitcast(g, jnp.bfloat16)
                lo, hi = plsc.unpack(gb, format=plsc.PackFormat.INTERLEAVED)
                plsc.addupdate_scatter(aggs[k], [row], lo * ew)
                plsc.addupdate_scatter(aggs[2 + k], [row], hi * ew)

    start(0, 0)

    def pair_body(cp, carry):
        for b in range(2):
            ci = cp * 2 + b
            wait(ci, b)

            @pl.when(ci + 1 < _NCHUNK)
            def _():
                start(ci + 1, 1 - b)

            process(b)
        return carry

    lax.fori_loop(0, _NCHUNK // 2, pair_body, 0)
    for k in range(2):
        olo = pl.multiple_of((w * 2 + k) * _N_PAD, _N_PAD)
        ohi = pl.multiple_of((w * 2 + 64 + k) * _N_PAD, _N_PAD)
        pltpu.sync_copy(aggs[k], agg_out.at[pl.ds(olo, _N_PAD)])
        pltpu.sync_copy(aggs[2 + k], agg_out.at[pl.ds(ohi, _N_PAD)])


def _sc_edge(hinp_flat, packed, ew):
    f = pl.kernel(
        _sc_edge_body,
        out_type=jax.ShapeDtypeStruct((_D * _N_PAD,), jnp.float32),
        mesh=_mesh,
        scratch_types=[pltpu.VMEM((_N_PAD,), jnp.int32)] * 2
        + [pltpu.VMEM((_N_PAD,), jnp.float32)] * 4
        + [pltpu.VMEM((_ECHUNK,), jnp.int32),
           pltpu.VMEM((_ECHUNK,), jnp.int32),
           pltpu.VMEM((_ECHUNK,), jnp.float32),
           pltpu.VMEM((_ECHUNK,), jnp.float32),
           pltpu.SemaphoreType.DMA,
           pltpu.SemaphoreType.DMA],
        compiler_params=_sc_params,
    )
    return f(hinp_flat, packed, ew)


def _pack_rows(hin):
    lo = lax.bitcast_convert_type(
        hin[:_D // 2].astype(jnp.bfloat16), jnp.uint16).astype(jnp.uint32)
    hi = lax.bitcast_convert_type(
        hin[_D // 2:].astype(jnp.bfloat16), jnp.uint16).astype(jnp.uint32)
    return lax.bitcast_convert_type(
        jnp.bitwise_or(jnp.left_shift(hi, 16), lo), jnp.int32)


def _tc0_body(pos_ref, h_ref, pe_ref, h0t_ref, hinp0_ref, pet_ref):
    ht = h_ref[...].T
    pet = pe_ref[...].T
    pet_ref[...] = pet
    h0t_ref[...] = ht + pet
    hinp0_ref[...] = _pack_rows(ht + pos_ref[0] * pet)


def _tc0(pos, h, pe):
    return pl.pallas_call(
        _tc0_body,
        grid=(_NBLK,),
        in_specs=[pl.BlockSpec(memory_space=pltpu.SMEM),
                  pl.BlockSpec((_CB, _D), lambda i: (i, 0)),
                  pl.BlockSpec((_CB, _D), lambda i: (i, 0))],
        out_specs=[pl.BlockSpec((_D, _CB), lambda i: (0, i)),
                   pl.BlockSpec((_D // 2, _CB), lambda i: (0, i)),
                   pl.BlockSpec((_D, _CB), lambda i: (0, i))],
        out_shape=[jax.ShapeDtypeStruct((_D, _N_PAD), jnp.float32),
                   jax.ShapeDtypeStruct((_D // 2, _N_PAD), jnp.int32),
                   jax.ShapeDtypeStruct((_D, _N_PAD), jnp.float32)],
    )(pos, h, pe)


def _mlp(agg, w1t_ref, b1_ref, w2t_ref, b2_ref, scale_ref, beta_ref):
    x = jnp.dot(w1t_ref[...], agg, preferred_element_type=jnp.float32)
    x = jnp.maximum(x + b1_ref[...], 0.0)
    x = jnp.dot(w2t_ref[...], x, preferred_element_type=jnp.float32)
    x = (x + b2_ref[...]) * scale_ref[...] + beta_ref[...]
    return jnp.where(x >= 0.0, x, 0.01 * x)


def _tc1_body(pos_ref, agg_ref, pe_ref, w1t_ref, b1_ref, w2t_ref, b2_ref,
              scale_ref, beta_ref, h1t_ref, hinp1_ref):
    h = _mlp(agg_ref[...], w1t_ref, b1_ref, w2t_ref, b2_ref, scale_ref,
             beta_ref)
    h1t_ref[...] = h
    hinp1_ref[...] = _pack_rows(h + pos_ref[0] * pe_ref[...])


def _tc1(pos1, aggt, pet, w1t, b1, w2t, b2, scale, beta):
    wspec = pl.BlockSpec((_D, _D), lambda i: (0, 0))
    bspec = pl.BlockSpec((_D, 1), lambda i: (0, 0))
    return pl.pallas_call(
        _tc1_body,
        grid=(_NBLK,),
        in_specs=[pl.BlockSpec(memory_space=pltpu.SMEM),
                  pl.BlockSpec((_D, _CB), lambda i: (0, i)),
                  pl.BlockSpec((_D, _CB), lambda i: (0, i)),
                  wspec, bspec, wspec, bspec, bspec, bspec],
        out_specs=[pl.BlockSpec((_D, _CB), lambda i: (0, i)),
                   pl.BlockSpec((_D // 2, _CB), lambda i: (0, i))],
        out_shape=[jax.ShapeDtypeStruct((_D, _N_PAD), jnp.float32),
                   jax.ShapeDtypeStruct((_D // 2, _N_PAD), jnp.int32)],
    )(pos1, aggt, pet, w1t, b1, w2t, b2, scale, beta)


def _tc2_body(attgp_ref, attb_ref, agg_ref, h0_ref, h1_ref, gid_ref,
              w1t_ref, b1_ref, w2t_ref, b2_ref, scale_ref, beta_ref,
              attw_ref, p0, p1, p2, r0, r1, r2):
    i = pl.program_id(0)
    pouts = [p0, p1, p2]
    routs = [r0, r1, r2]

    @pl.when(i == 0)
    def _():
        for l in range(3):
            pouts[l][...] = jnp.zeros((_D, _N_GRAPHS), jnp.float32)
            routs[l][...] = jnp.zeros((1, _N_GRAPHS), jnp.float32)

    h2 = _mlp(agg_ref[...], w1t_ref, b1_ref, w2t_ref, b2_ref, scale_ref,
              beta_ref)
    hs = [h0_ref[...], h1_ref[...], h2]
    gid = gid_ref[...]
    giota = lax.broadcasted_iota(jnp.int32, (_CB, _N_GRAPHS), 1)
    niota = lax.broadcasted_iota(jnp.int32, (_CB, 1), 0) + i * _CB
    onehot = jnp.where((gid == giota) & (niota < _N_NODES), 1.0, 0.0)
    e = jnp.ones((1, _CB), jnp.float32)
    for l in range(3):
        hl = hs[l]
        logits = (jnp.sum(hl * attw_ref[:, l:l + 1], axis=0, keepdims=True)
                  + e * attgp_ref[l] + attb_ref[l])
        lk = jnp.where(logits >= 0.0, logits, 0.01 * logits)
        e = jnp.exp(lk * (-1.0 / 20.0))
        pouts[l][...] += jnp.dot(hl * e, onehot,
                                 preferred_element_type=jnp.float32)
        routs[l][...] += jnp.dot(e, onehot,
                                 preferred_element_type=jnp.float32)


def _tc2(attgp, attb, aggt, h0t, h1t, gid2d, w1t, b1, w2t, b2, scale, beta,
         attw):
    wspec = pl.BlockSpec((_D, _D), lambda i: (0, 0))
    bspec = pl.BlockSpec((_D, 1), lambda i: (0, 0))
    cspec = pl.BlockSpec((_D, _CB), lambda i: (0, i))
    return pl.pallas_call(
        _tc2_body,
        grid=(_NBLK,),
        in_specs=[pl.BlockSpec(memory_space=pltpu.SMEM),
                  pl.BlockSpec(memory_space=pltpu.SMEM),
                  cspec, cspec, cspec,
                  pl.BlockSpec((_CB, 1), lambda i: (i, 0)),
                  wspec, bspec, wspec, bspec, bspec, bspec,
                  pl.BlockSpec((_D, 3), lambda i: (0, 0))],
        out_specs=[pl.BlockSpec((_D, _N_GRAPHS), lambda i: (0, 0))] * 3
        + [pl.BlockSpec((1, _N_GRAPHS), lambda i: (0, 0))] * 3,
        out_shape=[jax.ShapeDtypeStruct((_D, _N_GRAPHS), jnp.float32)] * 3
        + [jax.ShapeDtypeStruct((1, _N_GRAPHS), jnp.float32)] * 3,
    )(attgp, attb, aggt, h0t, h1t, gid2d, w1t, b1, w2t, b2, scale, beta, attw)


def _tcf_body(p0, p1, p2, r0, r1, r2, predwt_ref, predb_ref, score_ref):
    acc = jnp.broadcast_to(predb_ref[...], (_OUT_DIM, _N_GRAPHS))
    ps = [p0, p1, p2]
    rs = [r0, r1, r2]
    for l in range(3):
        pooled = ps[l][...] / rs[l][...]
        acc = acc + jnp.dot(predwt_ref[l], pooled,
                            preferred_element_type=jnp.float32)
    score_ref[...] = acc.T


def _tcf(p0, p1, p2, r0, r1, r2, predwt, predb_sum):
    return pl.pallas_call(
        _tcf_body,
        out_shape=jax.ShapeDtypeStruct((_N_GRAPHS, _OUT_DIM), jnp.float32),
    )(p0, p1, p2, r0, r1, r2, predwt, predb_sum)


def kernel(word_vectors, node_ids, positions, edge_index, edge_weight,
           graph_ids, pos, W_mlp, b_mlp, bn_gamma, bn_beta, att_w, att_b,
           pred_w, pred_b, pos_emb):
    f32 = jnp.float32
    npad = _N_PAD - _N_NODES
    ids1d = jnp.concatenate(
        [node_ids.astype(jnp.int32), jnp.zeros((npad,), jnp.int32)])
    posn1d = jnp.concatenate(
        [positions.astype(jnp.int32), jnp.zeros((npad,), jnp.int32)])
    packed = jnp.bitwise_or(
        jnp.left_shift(edge_index[0].astype(jnp.int32), 14),
        edge_index[1].astype(jnp.int32))
    gid2d = jnp.concatenate(
        [graph_ids.astype(jnp.int32),
         jnp.full((npad,), _N_GRAPHS - 1, jnp.int32)]
    ).reshape(_N_PAD, 1)

    inv = 1.0 / jnp.sqrt(jnp.asarray(1.0 + 1e-5, f32))
    w1t = [W_mlp[l, 0].T for l in range(2)]
    w2t = [W_mlp[l, 1].T for l in range(2)]
    b1 = [b_mlp[l, 0][:, None] for l in range(2)]
    b2 = [b_mlp[l, 1][:, None] for l in range(2)]
    scale = [(bn_gamma[l] * inv)[:, None] for l in range(2)]
    beta = [bn_beta[l][:, None] for l in range(2)]
    attw = att_w[:, :_D, 0].T
    attgp = att_w[:, _D, 0]
    attb = att_b[:, 0]
    predwt = jnp.transpose(pred_w, (0, 2, 1))
    predb_sum = jnp.sum(pred_b, axis=0)[:, None]

    h, pe = _sc_gather(word_vectors, ids1d, pos_emb, posn1d)
    h0t, hinp0, pet = _tc0(pos, h, pe)
    agg0t = _sc_edge(hinp0.reshape(-1), packed, edge_weight).reshape(_D, _N_PAD)
    h1t, hinp1 = _tc1(pos[1:2], agg0t, pet, w1t[0], b1[0], w2t[0], b2[0],
                      scale[0], beta[0])
    agg1t = _sc_edge(hinp1.reshape(-1), packed, edge_weight).reshape(_D, _N_PAD)
    p0, p1, p2, r0, r1, r2 = _tc2(attgp, attb, agg1t, h0t, h1t, gid2d,
                                  w1t[1], b1[1], w2t[1], b2[1], scale[1],
                                  beta[1], attw)
    return _tcf(p0, p1, p2, r0, r1, r2, predwt, predb_sum)
